# Initial kernel scaffold; baseline (speedup 1.0000x reference)
import jax
import jax.numpy as jnp
from jax.experimental import pallas as pl


def kernel(x, edge_index, W1, b1, W2, b2):
    raise NotImplementedError("write your pallas kernel here")


# speedup vs baseline: 33.3115x; 33.3115x over previous
import functools

import jax
import jax.numpy as jnp
from jax import lax
from jax.experimental import pallas as pl
from jax.experimental.pallas import tpu as pltpu
from jax.experimental.pallas import tpu_sc as plsc

N = 10000
E = 320000
F = 16
NC, NS = 2, 16
NW = NC * NS
BLK = 128
BPW = 80
PE = NW * BPW * BLK
ACC_ROWS = 10112
RPS = ACC_ROWS // NS
K = 8

_mesh = plsc.VectorSubcoreMesh(core_axis_name="c", subcore_axis_name="s")


def _edge_pass_body(gather, ht_hbm, row2d_hbm, col2d_hbm, out_hbm,
                    rows_v, cols_v, msg_v, zbuf_v, acc, gsem, ssem):
    cid = lax.axis_index("c")
    sid = lax.axis_index("s")
    wid = sid * NC + cid

    def _z(i, _):
        zbuf_v[i] = jnp.zeros((F,), jnp.float32)
        return 0
    lax.fori_loop(0, RPS, _z, 0)
    pltpu.sync_copy(zbuf_v, acc.at[pl.ds(sid * RPS, RPS)])
    plsc.subcore_barrier()

    base = wid * BPW
    pltpu.sync_copy(col2d_hbm.at[pl.ds(base, BPW)], cols_v)
    if gather:
        pltpu.sync_copy(row2d_hbm.at[pl.ds(base, BPW)], rows_v)
    else:
        def _o(i, _):
            msg_v[0, i] = jnp.ones((F,), jnp.float32)
            return 0
        lax.fori_loop(0, BLK, _o, 0)

    def _grp(g, _):
        j0 = g * K
        if gather:
            cps = [pltpu.async_copy(ht_hbm.at[rows_v.at[j0 + k]],
                                    msg_v.at[k], gsem)
                   for k in range(K)]
            for cp in cps:
                cp.wait()
            scs = [pltpu.async_copy(msg_v.at[k], acc.at[cols_v.at[j0 + k]],
                                    ssem, add=True)
                   for k in range(K)]
        else:
            scs = [pltpu.async_copy(msg_v.at[0], acc.at[cols_v.at[j0 + k]],
                                    ssem, add=True)
                   for k in range(K)]
        for cp in scs:
            cp.wait()
        return 0

    lax.fori_loop(0, BPW // K, _grp, 0)
    plsc.subcore_barrier()

    pltpu.sync_copy(acc.at[pl.ds(sid * RPS, RPS)], zbuf_v)
    pltpu.sync_copy(zbuf_v, out_hbm.at[cid].at[pl.ds(sid * RPS, RPS)])


def _make_edge_pass(gather):
    body = functools.partial(_edge_pass_body, gather)
    return pl.kernel(
        body,
        out_type=jax.ShapeDtypeStruct((NC, ACC_ROWS, F), jnp.float32),
        mesh=_mesh,
        scratch_types=[
            pltpu.VMEM((BPW, BLK), jnp.int32),
            pltpu.VMEM((BPW, BLK), jnp.int32),
            pltpu.VMEM((K, BLK, F), jnp.float32),
            pltpu.VMEM((RPS, F), jnp.float32),
            pltpu.VMEM_SHARED((ACC_ROWS, F), jnp.float32),
            pltpu.SemaphoreType.DMA,
            pltpu.SemaphoreType.DMA,
        ],
        compiler_params=pltpu.CompilerParams(use_tc_tiling_on_sc=False),
    )


_deg_pass = _make_edge_pass(False)
_msg_pass = _make_edge_pass(True)


def _tc1(x_ref, w_ref, p0_ref, p1_ref, dinv_ref, ht_ref):
    deg = p0_ref[...] + p1_ref[...] + 1.0
    dinv = lax.rsqrt(deg)
    dinv_ref[...] = dinv
    ht_ref[...] = dinv * jnp.dot(x_ref[...], w_ref[...],
                                 preferred_element_type=jnp.float32)


def _tc2(p0_ref, p1_ref, ht_ref, dinv_ref, w_ref, b_ref, out_ref):
    z = dinv_ref[...] * (p0_ref[...] + p1_ref[...] + ht_ref[...]) + b_ref[...]
    h = jnp.maximum(z, 0.0)
    out_ref[...] = dinv_ref[...] * jnp.dot(h, w_ref[...],
                                           preferred_element_type=jnp.float32)


def _tc3(p0_ref, p1_ref, ht_ref, dinv_ref, b_ref, out_ref):
    out_ref[...] = (dinv_ref[...] * (p0_ref[...] + p1_ref[...] + ht_ref[...])
                    + b_ref[...])


_f32 = jnp.float32
_nf = jax.ShapeDtypeStruct((N, F), _f32)


def kernel(x, edge_index, W1, b1, W2, b2):
    ei = edge_index.astype(jnp.int32)
    pad_r = jnp.zeros((PE - E,), jnp.int32)
    pad_c = jnp.full((PE - E,), N, jnp.int32)
    row2d = jnp.concatenate([ei[0], pad_r]).reshape(PE // BLK, BLK)
    col2d = jnp.concatenate([ei[1], pad_c]).reshape(PE // BLK, BLK)
    dummy = jnp.zeros((1, F), _f32)

    degp = _deg_pass(dummy, row2d, col2d)
    d0, d1 = degp[0, :N], degp[1, :N]
    dinv, ht1 = pl.pallas_call(
        _tc1, out_shape=(_nf, _nf))(x, W1, d0, d1)

    a1 = _msg_pass(ht1, row2d, col2d)
    ht2 = pl.pallas_call(_tc2, out_shape=_nf)(
        a1[0, :N], a1[1, :N], ht1, dinv, W2, b1.reshape(1, F))

    a2 = _msg_pass(ht2, row2d, col2d)
    out = pl.pallas_call(_tc3, out_shape=_nf)(
        a2[0, :N], a2[1, :N], ht2, dinv, b2.reshape(1, F))
    return out
